# baseline (speedup 1.0000x reference)
import functools

import jax
import jax.numpy as jnp
from jax import lax
from jax.experimental import pallas as pl
from jax.experimental.pallas import tpu as pltpu
from jax.experimental.pallas import tpu_sc as plsc

N = 10000
E = 320000
D = 128
L = 16

_info = plsc.get_sparse_core_info()
NC = _info.num_cores
NS = _info.num_subcores
NW = NC * NS
EPW = E // NW


def _sc_body(nt_hbm, comb_hbm, traj_hbm, out_hbm, comb_v, traj_v, nt_v, acc_v):
    wid = lax.axis_index("s") * NC + lax.axis_index("c")
    base = wid * EPW
    pltpu.sync_copy(comb_hbm.at[pl.ds(base, EPW)], comb_v)
    pltpu.sync_copy(traj_hbm.at[pl.ds(base, EPW)], traj_v)
    pltpu.sync_copy(nt_hbm, nt_v)

    zeros = jnp.zeros((L,), jnp.float32)

    @plsc.parallel_loop(0, N, L, unroll=1)
    def _zero(i):
        acc_v[0, pl.ds(i, L)] = zeros
        acc_v[1, pl.ds(i, L)] = zeros
        acc_v[2, pl.ds(i, L)] = zeros
        acc_v[3, pl.ds(i, L)] = zeros

    ones = jnp.ones((L,), jnp.float32)
    threes = jnp.full((L,), 3, jnp.int32)

    @plsc.parallel_loop(0, EPW, L, unroll=8)
    def _edges(i):
        c = comb_v[pl.ds(i, L)]
        t = traj_v[pl.ds(i, L)]
        s = lax.shift_right_logical(c, 14)
        d = lax.bitwise_and(c, 16383)
        ty = plsc.load_gather(nt_v, [s])
        plsc.addupdate_scatter(acc_v, [ty, d], ones)
        plsc.addupdate_scatter(acc_v, [threes, d], t)

    pltpu.sync_copy(acc_v, out_hbm.at[pl.ds(wid * 4, 4)])


def _sc_scatter(nt, comb, traj):
    mesh = plsc.VectorSubcoreMesh(core_axis_name="c", subcore_axis_name="s")
    fn = functools.partial(
        pl.kernel,
        mesh=mesh,
        out_type=jax.ShapeDtypeStruct((NW * 4, N), jnp.float32),
        scratch_types=[
            pltpu.VMEM((EPW,), jnp.int32),
            pltpu.VMEM((EPW,), jnp.float32),
            pltpu.VMEM((N,), jnp.int32),
            pltpu.VMEM((4, N), jnp.float32),
        ],
        compiler_params=pltpu.CompilerParams(needs_layout_passes=False),
    )(_sc_body)
    return fn(nt, comb, traj)


def _pack_body(ei_ref, out_ref):
    s = ei_ref[0:1, :]
    d = ei_ref[1:2, :]
    out_ref[...] = (s * 16384 + d).reshape(E)


def _pack_edges(ei):
    return pl.pallas_call(
        _pack_body,
        out_shape=jax.ShapeDtypeStruct((E,), jnp.int32),
    )(ei)


def _tc_body(part_ref, nt_ref, Wes_ref, Wed_ref, wet_ref, Wn3_ref, WnD_ref,
             be_ref, bn_ref, out_ref):
    red = jnp.sum(part_ref[...].reshape(NW, 4, N), axis=0)
    nt = nt_ref[...]
    tval = lax.broadcasted_iota(jnp.int32, (3, N), 0)
    nf = (tval == nt).astype(jnp.float32)
    deg = red[0:1] + red[1:2] + red[2:3]
    U = jnp.concatenate([red[0:3], deg, red[3:4], nf, deg * nf], axis=0)

    WnD = WnD_ref[...]
    cdim = (((0,), (1,)), ((), ()))
    A = lax.dot_general(Wes_ref[...], WnD, cdim,
                        preferred_element_type=jnp.float32)
    B = lax.dot_general(Wed_ref[...], WnD, cdim,
                        preferred_element_type=jnp.float32)
    c = lax.dot_general(wet_ref[...], WnD, cdim,
                        preferred_element_type=jnp.float32)
    d = lax.dot_general(be_ref[...], WnD, (((1,), (1,)), ((), ())),
                        preferred_element_type=jnp.float32)
    eye3 = (lax.broadcasted_iota(jnp.int32, (3, 3), 0) ==
            lax.broadcasted_iota(jnp.int32, (3, 3), 1)).astype(jnp.float32)
    Wn3T = lax.dot_general(eye3, Wn3_ref[...], (((1,), (1,)), ((), ())),
                           preferred_element_type=jnp.float32)
    W = jnp.concatenate([A, d, c, Wn3T, B], axis=0)

    out = lax.dot_general(U, W, (((0,), (0,)), ((), ())),
                          preferred_element_type=jnp.float32)
    out_ref[...] = out + bn_ref[...]


def kernel(node_type, edge_index, traj, We, be, Wn, bn):
    nt = node_type.astype(jnp.int32)
    ei = edge_index.astype(jnp.int32)
    comb = _pack_edges(ei)

    part = _sc_scatter(nt, comb, traj)

    out = pl.pallas_call(
        _tc_body,
        out_shape=jax.ShapeDtypeStruct((N, D), jnp.float32),
    )(part, nt.reshape(1, N), We[:, 0:3], We[:, 3:6], We[:, 6:7],
      Wn[:, 0:3], Wn[:, 3:], be.reshape(1, D), bn.reshape(1, D))
    return out


# speedup vs baseline: 1.1011x; 1.1011x over previous
import functools

import jax
import jax.numpy as jnp
from jax import lax
from jax.experimental import pallas as pl
from jax.experimental.pallas import tpu as pltpu
from jax.experimental.pallas import tpu_sc as plsc

N = 10000
E = 320000
D = 128
L = 16

NP = 10112

_info = plsc.get_sparse_core_info()
NC = _info.num_cores
NS = _info.num_subcores
NW = NC * NS
EPW = E // NW


def _sc_body(nt_hbm, comb_hbm, traj_hbm, out_hbm,
             comb_v, traj_v, nt_v, acc_v, sem):
    cid = lax.axis_index("c")
    sid = lax.axis_index("s")
    wid = sid * NC + cid
    base = wid * EPW
    d0 = pltpu.async_copy(comb_hbm.at[pl.ds(base, EPW)], comb_v, sem)
    d1 = pltpu.async_copy(traj_hbm.at[pl.ds(base, EPW)], traj_v, sem)
    d2 = pltpu.async_copy(nt_hbm, nt_v, sem)

    zeros = jnp.zeros((L,), jnp.float32)

    @plsc.parallel_loop(0, NP, L, unroll=4)
    def _zero(i):
        acc_v[0, pl.ds(i, L)] = zeros
        acc_v[1, pl.ds(i, L)] = zeros
        acc_v[2, pl.ds(i, L)] = zeros
        acc_v[3, pl.ds(i, L)] = zeros

    d0.wait()
    d1.wait()
    d2.wait()

    ones = jnp.ones((L,), jnp.float32)
    threes = jnp.full((L,), 3, jnp.int32)

    @plsc.parallel_loop(0, EPW, L, unroll=4)
    def _edges(i):
        c = comb_v[pl.ds(i, L)]
        t = traj_v[pl.ds(i, L)]
        s = lax.shift_right_logical(c, 14)
        d = lax.bitwise_and(c, 16383)
        ty = plsc.load_gather(nt_v, [s])
        plsc.addupdate_scatter(acc_v, [ty, d], ones)
        plsc.addupdate_scatter(acc_v, [threes, d], t)

    pltpu.sync_copy(acc_v, out_hbm.at[pl.ds(wid * 4, 4)])


def _sc_scatter(nt, comb, traj):
    mesh = plsc.VectorSubcoreMesh(core_axis_name="c", subcore_axis_name="s")
    fn = functools.partial(
        pl.kernel,
        mesh=mesh,
        out_type=jax.ShapeDtypeStruct((NW * 4, NP), jnp.float32),
        scratch_types=[
            pltpu.VMEM((EPW,), jnp.int32),
            pltpu.VMEM((EPW,), jnp.float32),
            pltpu.VMEM((N,), jnp.int32),
            pltpu.VMEM((4, NP), jnp.float32),
            pltpu.SemaphoreType.DMA,
        ],
        compiler_params=pltpu.CompilerParams(needs_layout_passes=False),
    )(_sc_body)
    return fn(nt, comb, traj)


def _pack_body(ei_ref, out_ref):
    s = ei_ref[0:1, :]
    d = ei_ref[1:2, :]
    out_ref[...] = (s * 16384 + d).reshape(E)


def _pack_edges(ei):
    return pl.pallas_call(
        _pack_body,
        out_shape=jax.ShapeDtypeStruct((E,), jnp.int32),
    )(ei)


def _tc_body(part_ref, nt_ref, Wes_ref, Wed_ref, wet_ref, Wn3_ref, WnD_ref,
             be_ref, bn_ref, out_ref):
    red = jnp.sum(part_ref[...].reshape(NW, 4, NP), axis=0)[:, :N]
    nt = nt_ref[...]
    tval = lax.broadcasted_iota(jnp.int32, (3, N), 0)
    nf = (tval == nt).astype(jnp.float32)
    deg = red[0:1] + red[1:2] + red[2:3]
    U = jnp.concatenate([red[0:3], deg, red[3:4], nf, deg * nf], axis=0)

    WnD = WnD_ref[...]
    cdim = (((0,), (1,)), ((), ()))
    A = lax.dot_general(Wes_ref[...], WnD, cdim,
                        preferred_element_type=jnp.float32)
    B = lax.dot_general(Wed_ref[...], WnD, cdim,
                        preferred_element_type=jnp.float32)
    c = lax.dot_general(wet_ref[...], WnD, cdim,
                        preferred_element_type=jnp.float32)
    d = lax.dot_general(be_ref[...], WnD, (((1,), (1,)), ((), ())),
                        preferred_element_type=jnp.float32)
    eye3 = (lax.broadcasted_iota(jnp.int32, (3, 3), 0) ==
            lax.broadcasted_iota(jnp.int32, (3, 3), 1)).astype(jnp.float32)
    Wn3T = lax.dot_general(eye3, Wn3_ref[...], (((1,), (1,)), ((), ())),
                           preferred_element_type=jnp.float32)
    W = jnp.concatenate([A, d, c, Wn3T, B], axis=0)

    out = lax.dot_general(U, W, (((0,), (0,)), ((), ())),
                          preferred_element_type=jnp.float32)
    out_ref[...] = out + bn_ref[...]


def kernel(node_type, edge_index, traj, We, be, Wn, bn):
    nt = node_type.astype(jnp.int32)
    ei = edge_index.astype(jnp.int32)
    comb = _pack_edges(ei)

    part = _sc_scatter(nt, comb, traj)

    out = pl.pallas_call(
        _tc_body,
        out_shape=jax.ShapeDtypeStruct((N, D), jnp.float32),
    )(part, nt.reshape(1, N), We[:, 0:3], We[:, 3:6], We[:, 6:7],
      Wn[:, 0:3], Wn[:, 3:], be.reshape(1, D), bn.reshape(1, D))
    return out
